# baseline (speedup 1.0000x reference)
import functools

import jax
import jax.numpy as jnp
from jax import lax
from jax.experimental import pallas as pl
from jax.experimental.pallas import tpu as pltpu
from jax.experimental.pallas import tpu_sc as plsc

_NUM_CORES = 1
_NUM_SUBCORES = 16
_NUM_WORKERS = _NUM_CORES * _NUM_SUBCORES
_LANES = 16


def _gather_body(n_per_worker, values_hbm, t_hbm, out_hbm,
                 table_v, idx_v, out_v, sem_t, sem_i0, sem_i1, sem_o):
    wid = lax.axis_index("s") * _NUM_CORES + lax.axis_index("c")
    base = wid * n_per_worker
    half = n_per_worker // 2

    cp_t = pltpu.async_copy(values_hbm, table_v, sem_t)
    cp_i0 = pltpu.async_copy(t_hbm.at[pl.ds(base, half)],
                             idx_v.at[pl.ds(0, half)], sem_i0)
    cp_i1 = pltpu.async_copy(t_hbm.at[pl.ds(base + half, half)],
                             idx_v.at[pl.ds(half, half)], sem_i1)
    cp_t.wait()
    cp_i0.wait()

    @plsc.parallel_loop(0, half // _LANES, unroll=8)
    def _(i):
        idx = idx_v[pl.ds(i * _LANES, _LANES)]
        out_v[pl.ds(i * _LANES, _LANES)] = plsc.load_gather(table_v, [idx])

    cp_o0 = pltpu.async_copy(out_v.at[pl.ds(0, half)],
                             out_hbm.at[pl.ds(base, half)], sem_o)
    cp_i1.wait()

    @plsc.parallel_loop(0, half // _LANES, unroll=8)
    def _(i):
        idx = idx_v[pl.ds(half + i * _LANES, _LANES)]
        out_v[pl.ds(half + i * _LANES, _LANES)] = plsc.load_gather(
            table_v, [idx])

    cp_o1 = pltpu.async_copy(out_v.at[pl.ds(half, half)],
                             out_hbm.at[pl.ds(base + half, half)], sem_o)
    cp_o0.wait()
    cp_o1.wait()


@jax.jit
def kernel(values, t):
    num_t = values.shape[0]
    batch = t.shape[0]
    n_per_worker = batch // _NUM_WORKERS

    mesh = plsc.VectorSubcoreMesh(
        core_axis_name="c", subcore_axis_name="s",
        num_cores=_NUM_CORES, num_subcores=_NUM_SUBCORES)
    gather = pl.kernel(
        functools.partial(_gather_body, n_per_worker),
        out_type=jax.ShapeDtypeStruct((batch,), jnp.float32),
        mesh=mesh,
        scratch_types=[
            pltpu.VMEM((num_t,), jnp.float32),
            pltpu.VMEM((n_per_worker,), jnp.int32),
            pltpu.VMEM((n_per_worker,), jnp.float32),
            pltpu.SemaphoreType.DMA,
            pltpu.SemaphoreType.DMA,
            pltpu.SemaphoreType.DMA,
            pltpu.SemaphoreType.DMA,
        ],
        compiler_params=pltpu.CompilerParams(needs_layout_passes=False),
    )
    out = gather(values, t)
    return out.reshape(batch, 1, 1, 1)
